# baseline (speedup 1.0000x reference)
import functools

import jax
import jax.numpy as jnp
from jax import lax
from jax.experimental import pallas as pl
from jax.experimental.pallas import tpu as pltpu
from jax.experimental.pallas import tpu_sc as plsc

N_USER = 50000
N_ITEM = 10000
D = 128
E = 320000

NC = 2
NS = 16
NW = NC * NS
ROWS_PER_W = 80
E_PAD = NW * ROWS_PER_W * 128
G = 8

N_ITEM_PAD = 10240
N_USER_PAD = 50048

U2I_SPLIT = (160, 0)
I2U_SPLIT = (112, 48)


def _core_base_rows(c, s, split):
  r0, r1 = split
  base = jnp.where(c == 0, s * r0, NS * r0 + s * r1)
  n_groups = jnp.where(c == 0, r0 // 16, r1 // 16)
  return base, n_groups


def _memset_zero(buf, width):
  z = jnp.zeros((16,), jnp.float32)

  @pl.loop(0, 128)
  def _row(i):
    for k in range(width // 16):
      buf[i, pl.ds(k * 16, 16)] = z


def _zero_stripe(acc, buf, row0, n_full, tail, width):
  @pl.loop(0, n_full)
  def _chunk(i):
    pltpu.sync_copy(buf, acc.at[pl.ds(row0 + i * 128, 128)])
  if tail:
    pltpu.sync_copy(buf.at[pl.ds(0, tail)],
                    acc.at[pl.ds(row0 + n_full * 128, tail)])

_MESH = plsc.VectorSubcoreMesh(
    core_axis_name="c", subcore_axis_name="s", num_cores=NC, num_subcores=NS
)


_GATHER_DNUMS = lax.GatherDimensionNumbers(
    offset_dims=(), collapsed_slice_dims=(0,), start_index_map=(0,))


def _lane_bcast(v16, i):
  idx = jnp.broadcast_to(i, (16,)).astype(jnp.int32)
  return lax.gather(v16, idx[:, None], _GATHER_DNUMS, (1,),
                    mode=lax.GatherScatterMode.PROMISE_IN_BOUNDS)


def _scale_rows(rows_v, w_v, j, width):
  @pl.loop(0, 8)
  def _grp(l):
    w16 = w_v[j, pl.ds(l * 16, 16)]
    for t in range(16):
      ws = _lane_bcast(w16, t)
      e = l * 16 + t
      for k in range(width // 16):
        sl = pl.ds(k * 16, 16)
        rows_v[e, sl] = rows_v[e, sl] * ws


GG = 16


def _edge_pipeline(x_hbm, src_hbm, dst_hbm, w_hbm, base, n_groups,
                   src_v, dst_v, w_v, rows_a, rows_b, acc,
                   sem_a, sem_b, ssem_a, ssem_b, width):
  del ssem_a, ssem_b

  def gather(j, buf, sem):
    pltpu.async_copy(x_hbm.at[src_v.at[j]], buf, sem)

  def consume(j, buf, sem):
    pltpu.make_async_copy(x_hbm.at[src_v.at[j]], buf, sem).wait()
    _scale_rows(buf, w_v, j, width)
    pltpu.sync_copy(buf, acc.at[dst_v.at[j]], add=True)

  @pl.loop(0, n_groups)
  def _group(g):
    row0 = base + g * GG
    pltpu.sync_copy(src_hbm.at[pl.ds(row0, GG)], src_v)
    pltpu.sync_copy(dst_hbm.at[pl.ds(row0, GG)], dst_v)
    pltpu.sync_copy(w_hbm.at[pl.ds(row0, GG)], w_v)
    gather(0, rows_a, sem_a)

    @pl.loop(0, GG // 2)
    def _pair(m):
      ja = 2 * m
      jb = 2 * m + 1
      gather(jb, rows_b, sem_b)
      consume(ja, rows_a, sem_a)

      @pl.when(m < GG // 2 - 1)
      def _():
        gather(jb + 1, rows_a, sem_a)
      consume(jb, rows_b, sem_b)


def _sc_u2i_body(x_hbm, src_hbm, dst_hbm, w_hbm, out_hbm,
                 src_v, dst_v, w_v, rows_a, rows_b, acc,
                 sem_a, sem_b, ssem_a, ssem_b):
  c = lax.axis_index("c")
  s = lax.axis_index("s")
  stripe = N_ITEM_PAD // NS
  base, n_groups = _core_base_rows(c, s, U2I_SPLIT)
  _memset_zero(rows_b, D)
  _zero_stripe(acc, rows_b, s * stripe, stripe // 128, 0, D)
  plsc.subcore_barrier()

  _edge_pipeline(x_hbm, src_hbm, dst_hbm, w_hbm, base, n_groups,
                 src_v, dst_v, w_v, rows_a, rows_b, acc,
                 sem_a, sem_b, ssem_a, ssem_b, D)

  plsc.subcore_barrier()

  @pl.when(c == 0)
  def _():
    pltpu.sync_copy(acc.at[pl.ds(s * stripe, stripe)],
                    out_hbm.at[pl.ds(s * stripe, stripe)])


_sc_u2i = functools.partial(
    pl.kernel,
    out_type=jax.ShapeDtypeStruct((N_ITEM_PAD, D), jnp.float32),
    mesh=_MESH,
    scratch_types=[
        pltpu.VMEM((GG, 128), jnp.int32),
        pltpu.VMEM((GG, 128), jnp.int32),
        pltpu.VMEM((GG, 128), jnp.float32),
        pltpu.VMEM((128, D), jnp.float32),
        pltpu.VMEM((128, D), jnp.float32),
        pltpu.VMEM_SHARED((N_ITEM_PAD, D), jnp.float32),
        pltpu.SemaphoreType.DMA,
        pltpu.SemaphoreType.DMA,
        pltpu.SemaphoreType.DMA,
        pltpu.SemaphoreType.DMA,
    ],
)(_sc_u2i_body)


def _sc_i2u_body(x4_hbm, src_hbm, dst_hbm, w_hbm, out_hbm,
                 src_v, dst_v, w_v, rows_a, rows_b, acc,
                 sem_a, sem_b, ssem_a, ssem_b):
  c = lax.axis_index("c")
  s = lax.axis_index("s")
  stripe = N_USER_PAD // NS
  base, n_groups = _core_base_rows(c, s, I2U_SPLIT)

  for q in range(4):
    _memset_zero(rows_b, 32)
    _zero_stripe(acc, rows_b, s * stripe, stripe // 128, stripe % 128, 32)
    plsc.subcore_barrier()

    _edge_pipeline(x4_hbm.at[q], src_hbm, dst_hbm, w_hbm, base, n_groups,
                   src_v, dst_v, w_v, rows_a, rows_b, acc,
                   sem_a, sem_b, ssem_a, ssem_b, 32)

    plsc.subcore_barrier()
    pltpu.sync_copy(acc.at[pl.ds(s * stripe, stripe)],
                    out_hbm.at[c].at[pl.ds(s * stripe, stripe),
                                     pl.ds(q * 32, 32)])
    plsc.subcore_barrier()


_sc_i2u = functools.partial(
    pl.kernel,
    out_type=jax.ShapeDtypeStruct((NC, N_USER_PAD, D), jnp.float32),
    mesh=_MESH,
    scratch_types=[
        pltpu.VMEM((GG, 128), jnp.int32),
        pltpu.VMEM((GG, 128), jnp.int32),
        pltpu.VMEM((GG, 128), jnp.float32),
        pltpu.VMEM((128, 32), jnp.float32),
        pltpu.VMEM((128, 32), jnp.float32),
        pltpu.VMEM_SHARED((N_USER_PAD, 32), jnp.float32),
        pltpu.SemaphoreType.DMA,
        pltpu.SemaphoreType.DMA,
        pltpu.SemaphoreType.DMA,
        pltpu.SemaphoreType.DMA,
    ],
    compiler_params=pltpu.CompilerParams(use_tc_tiling_on_sc=False),
)(_sc_i2u_body)


def _tc_combine_body(p_ref, x_ref, wm_ref, ws_ref, b_ref, o_ref):
  if p_ref.shape[0] == 2:
    agg = p_ref[0] + p_ref[1]
  else:
    agg = p_ref[0]
  o_ref[...] = (
      jnp.dot(agg, wm_ref[...], preferred_element_type=jnp.float32)
      + jnp.dot(x_ref[...], ws_ref[...], preferred_element_type=jnp.float32)
      + b_ref[...]
  )


def _tc_combine(p, x, wm, ws, b, blk):
  n = x.shape[0]
  if p.ndim == 2:
    p = p.reshape(1, *p.shape)
  nparts = p.shape[0]
  return pl.pallas_call(
      _tc_combine_body,
      grid=(n // blk,),
      in_specs=[
          pl.BlockSpec((nparts, blk, D), lambda i: (0, i, 0)),
          pl.BlockSpec((blk, D), lambda i: (i, 0)),
          pl.BlockSpec((D, D), lambda i: (0, 0)),
          pl.BlockSpec((D, D), lambda i: (0, 0)),
          pl.BlockSpec((1, D), lambda i: (0, 0)),
      ],
      out_specs=pl.BlockSpec((blk, D), lambda i: (i, 0)),
      out_shape=jax.ShapeDtypeStruct((n, D), jnp.float32),
  )(p, x, wm, ws, b.reshape(1, D))


def _prep_edges(src, dst, w):
  pad = E_PAD - E
  src = jnp.concatenate([src.astype(jnp.int32), jnp.zeros((pad,), jnp.int32)])
  dst = jnp.concatenate([dst.astype(jnp.int32), jnp.zeros((pad,), jnp.int32)])
  w = jnp.concatenate([w, jnp.zeros((pad,), jnp.float32)])
  return src.reshape(-1, 128), dst.reshape(-1, 128), w.reshape(-1, 128)


def kernel(x_user, x_item, src_u2i, dst_u2i, edge_weight_u2i,
           src_i2u, dst_i2u, edge_weight_i2u,
           W_msg_u2i, W_self_u2i, b_u2i,
           W_msg_i2u, W_self_i2u, b_i2u):
  su, du, wu = _prep_edges(src_u2i, dst_u2i, edge_weight_u2i)
  si, di, wi = _prep_edges(src_i2u, dst_i2u, edge_weight_i2u)
  xi4 = x_item.reshape(N_ITEM, 4, 32).transpose(1, 0, 2)

  p_item = _sc_u2i(x_user, su, du, wu)
  p_user = _sc_i2u(xi4, si, di, wi)

  out_item = _tc_combine(p_item, x_item, W_msg_u2i, W_self_u2i, b_u2i, 2000)
  out_user = _tc_combine(p_user, x_user, W_msg_i2u, W_self_i2u, b_i2u, 2000)
  return (out_user, out_item)


# speedup vs baseline: 1.0899x; 1.0899x over previous
import functools

import jax
import jax.numpy as jnp
from jax import lax
from jax.experimental import pallas as pl
from jax.experimental.pallas import tpu as pltpu
from jax.experimental.pallas import tpu_sc as plsc

N_USER = 50000
N_ITEM = 10000
D = 128
E = 320000

NC = 2
NS = 16
NW = NC * NS
ROWS_PER_W = 80
E_PAD = NW * ROWS_PER_W * 128
G = 8

N_ITEM_PAD = 10240
N_USER_PAD = 50048

U2I_SPLIT = (112, 48)
I2U_SPLIT = (96, 64)


def _core_base_rows(c, s, split):
  r0, r1 = split
  base = jnp.where(c == 0, s * r0, NS * r0 + s * r1)
  n_groups = jnp.where(c == 0, r0 // 16, r1 // 16)
  return base, n_groups


def _memset_zero(buf, width):
  z = jnp.zeros((16,), jnp.float32)

  @pl.loop(0, 128)
  def _row(i):
    for k in range(width // 16):
      buf[i, pl.ds(k * 16, 16)] = z


def _zero_stripe(acc, buf, row0, n_full, tail, width):
  @pl.loop(0, n_full)
  def _chunk(i):
    pltpu.sync_copy(buf, acc.at[pl.ds(row0 + i * 128, 128)])
  if tail:
    pltpu.sync_copy(buf.at[pl.ds(0, tail)],
                    acc.at[pl.ds(row0 + n_full * 128, tail)])

_MESH = plsc.VectorSubcoreMesh(
    core_axis_name="c", subcore_axis_name="s", num_cores=NC, num_subcores=NS
)


_GATHER_DNUMS = lax.GatherDimensionNumbers(
    offset_dims=(), collapsed_slice_dims=(0,), start_index_map=(0,))


def _lane_bcast(v16, i):
  idx = jnp.broadcast_to(i, (16,)).astype(jnp.int32)
  return lax.gather(v16, idx[:, None], _GATHER_DNUMS, (1,),
                    mode=lax.GatherScatterMode.PROMISE_IN_BOUNDS)


def _scale_rows(rows_v, w_v, j, width):
  @pl.loop(0, 8)
  def _grp(l):
    w16 = w_v[j, pl.ds(l * 16, 16)]
    for t in range(16):
      ws = _lane_bcast(w16, t)
      e = l * 16 + t
      for k in range(width // 16):
        sl = pl.ds(k * 16, 16)
        rows_v[e, sl] = rows_v[e, sl] * ws


GG = 16


def _edge_pipeline(x_hbm, src_hbm, dst_hbm, w_hbm, base, n_groups,
                   src_v, dst_v, w_v, rows_a, rows_b, acc,
                   sem_a, sem_b, ssem_a, ssem_b, width):
  del ssem_a, ssem_b

  def gather(j, buf, sem):
    pltpu.async_copy(x_hbm.at[src_v.at[j]], buf, sem)

  def consume(j, buf, sem):
    pltpu.make_async_copy(x_hbm.at[src_v.at[j]], buf, sem).wait()
    _scale_rows(buf, w_v, j, width)
    pltpu.sync_copy(buf, acc.at[dst_v.at[j]], add=True)

  @pl.loop(0, n_groups)
  def _group(g):
    row0 = base + g * GG
    pltpu.sync_copy(src_hbm.at[pl.ds(row0, GG)], src_v)
    pltpu.sync_copy(dst_hbm.at[pl.ds(row0, GG)], dst_v)
    pltpu.sync_copy(w_hbm.at[pl.ds(row0, GG)], w_v)
    gather(0, rows_a, sem_a)

    @pl.loop(0, GG // 2)
    def _pair(m):
      ja = 2 * m
      jb = 2 * m + 1
      gather(jb, rows_b, sem_b)
      consume(ja, rows_a, sem_a)

      @pl.when(m < GG // 2 - 1)
      def _():
        gather(jb + 1, rows_a, sem_a)
      consume(jb, rows_b, sem_b)


def _sc_u2i_body(x_hbm, src_hbm, dst_hbm, w_hbm, out_hbm,
                 src_v, dst_v, w_v, rows_a, rows_b, acc,
                 sem_a, sem_b, ssem_a, ssem_b):
  c = lax.axis_index("c")
  s = lax.axis_index("s")
  stripe = N_ITEM_PAD // NS
  base, n_groups = _core_base_rows(c, s, U2I_SPLIT)
  _memset_zero(rows_b, D)
  _zero_stripe(acc, rows_b, s * stripe, stripe // 128, 0, D)
  plsc.subcore_barrier()

  _edge_pipeline(x_hbm, src_hbm, dst_hbm, w_hbm, base, n_groups,
                 src_v, dst_v, w_v, rows_a, rows_b, acc,
                 sem_a, sem_b, ssem_a, ssem_b, D)

  plsc.subcore_barrier()
  pltpu.sync_copy(acc.at[pl.ds(s * stripe, stripe)],
                  out_hbm.at[c].at[pl.ds(s * stripe, stripe)])


_sc_u2i = functools.partial(
    pl.kernel,
    out_type=jax.ShapeDtypeStruct((NC, N_ITEM_PAD, D), jnp.float32),
    mesh=_MESH,
    scratch_types=[
        pltpu.VMEM((GG, 128), jnp.int32),
        pltpu.VMEM((GG, 128), jnp.int32),
        pltpu.VMEM((GG, 128), jnp.float32),
        pltpu.VMEM((128, D), jnp.float32),
        pltpu.VMEM((128, D), jnp.float32),
        pltpu.VMEM_SHARED((N_ITEM_PAD, D), jnp.float32),
        pltpu.SemaphoreType.DMA,
        pltpu.SemaphoreType.DMA,
        pltpu.SemaphoreType.DMA,
        pltpu.SemaphoreType.DMA,
    ],
)(_sc_u2i_body)


def _sc_i2u_body(x4_hbm, src_hbm, dst_hbm, w_hbm, out_hbm,
                 src_v, dst_v, w_v, rows_a, rows_b, acc,
                 sem_a, sem_b, ssem_a, ssem_b):
  c = lax.axis_index("c")
  s = lax.axis_index("s")
  stripe = N_USER_PAD // NS
  base, n_groups = _core_base_rows(c, s, I2U_SPLIT)

  for q in range(4):
    _memset_zero(rows_b, 32)
    _zero_stripe(acc, rows_b, s * stripe, stripe // 128, stripe % 128, 32)
    plsc.subcore_barrier()

    _edge_pipeline(x4_hbm.at[q], src_hbm, dst_hbm, w_hbm, base, n_groups,
                   src_v, dst_v, w_v, rows_a, rows_b, acc,
                   sem_a, sem_b, ssem_a, ssem_b, 32)

    plsc.subcore_barrier()
    pltpu.sync_copy(acc.at[pl.ds(s * stripe, stripe)],
                    out_hbm.at[c].at[pl.ds(s * stripe, stripe),
                                     pl.ds(q * 32, 32)])
    plsc.subcore_barrier()


_sc_i2u = functools.partial(
    pl.kernel,
    out_type=jax.ShapeDtypeStruct((NC, N_USER_PAD, D), jnp.float32),
    mesh=_MESH,
    scratch_types=[
        pltpu.VMEM((GG, 128), jnp.int32),
        pltpu.VMEM((GG, 128), jnp.int32),
        pltpu.VMEM((GG, 128), jnp.float32),
        pltpu.VMEM((128, 32), jnp.float32),
        pltpu.VMEM((128, 32), jnp.float32),
        pltpu.VMEM_SHARED((N_USER_PAD, 32), jnp.float32),
        pltpu.SemaphoreType.DMA,
        pltpu.SemaphoreType.DMA,
        pltpu.SemaphoreType.DMA,
        pltpu.SemaphoreType.DMA,
    ],
    compiler_params=pltpu.CompilerParams(use_tc_tiling_on_sc=False),
)(_sc_i2u_body)


def _tc_combine_body(p_ref, x_ref, wm_ref, ws_ref, b_ref, o_ref):
  if p_ref.shape[0] == 2:
    agg = p_ref[0] + p_ref[1]
  else:
    agg = p_ref[0]
  o_ref[...] = (
      jnp.dot(agg, wm_ref[...], preferred_element_type=jnp.float32)
      + jnp.dot(x_ref[...], ws_ref[...], preferred_element_type=jnp.float32)
      + b_ref[...]
  )


def _tc_combine(p, x, wm, ws, b, blk):
  n = x.shape[0]
  if p.ndim == 2:
    p = p.reshape(1, *p.shape)
  nparts = p.shape[0]
  return pl.pallas_call(
      _tc_combine_body,
      grid=(n // blk,),
      in_specs=[
          pl.BlockSpec((nparts, blk, D), lambda i: (0, i, 0)),
          pl.BlockSpec((blk, D), lambda i: (i, 0)),
          pl.BlockSpec((D, D), lambda i: (0, 0)),
          pl.BlockSpec((D, D), lambda i: (0, 0)),
          pl.BlockSpec((1, D), lambda i: (0, 0)),
      ],
      out_specs=pl.BlockSpec((blk, D), lambda i: (i, 0)),
      out_shape=jax.ShapeDtypeStruct((n, D), jnp.float32),
  )(p, x, wm, ws, b.reshape(1, D))


def _prep_edges(src, dst, w):
  pad = E_PAD - E
  src = jnp.concatenate([src.astype(jnp.int32), jnp.zeros((pad,), jnp.int32)])
  dst = jnp.concatenate([dst.astype(jnp.int32), jnp.zeros((pad,), jnp.int32)])
  w = jnp.concatenate([w, jnp.zeros((pad,), jnp.float32)])
  return src.reshape(-1, 128), dst.reshape(-1, 128), w.reshape(-1, 128)


def kernel(x_user, x_item, src_u2i, dst_u2i, edge_weight_u2i,
           src_i2u, dst_i2u, edge_weight_i2u,
           W_msg_u2i, W_self_u2i, b_u2i,
           W_msg_i2u, W_self_i2u, b_i2u):
  su, du, wu = _prep_edges(src_u2i, dst_u2i, edge_weight_u2i)
  si, di, wi = _prep_edges(src_i2u, dst_i2u, edge_weight_i2u)
  xi4 = x_item.reshape(N_ITEM, 4, 32).transpose(1, 0, 2)

  p_item = _sc_u2i(x_user, su, du, wu)
  p_user = _sc_i2u(xi4, si, di, wi)

  out_item = _tc_combine(p_item, x_item, W_msg_u2i, W_self_u2i, b_u2i, 2000)
  out_user = _tc_combine(p_user, x_user, W_msg_i2u, W_self_i2u, b_i2u, 2000)
  return (out_user, out_item)


# speedup vs baseline: 1.1307x; 1.0375x over previous
import functools

import jax
import jax.numpy as jnp
from jax import lax
from jax.experimental import pallas as pl
from jax.experimental.pallas import tpu as pltpu
from jax.experimental.pallas import tpu_sc as plsc

N_USER = 50000
N_ITEM = 10000
D = 128
E = 320000

NC = 2
NS = 16
NW = NC * NS
ROWS_PER_W = 80
E_PAD = NW * ROWS_PER_W * 128
G = 8

N_ITEM_PAD = 10240
N_USER_PAD = 50048

U2I_SPLIT = (112, 48)
I2U_SPLIT = (112, 48)


def _core_base_rows(c, s, split):
  r0, r1 = split
  base = jnp.where(c == 0, s * r0, NS * r0 + s * r1)
  n_groups = jnp.where(c == 0, r0 // 16, r1 // 16)
  return base, n_groups


def _memset_zero(buf, width):
  z = jnp.zeros((16,), jnp.float32)

  @pl.loop(0, 128)
  def _row(i):
    for k in range(width // 16):
      buf[i, pl.ds(k * 16, 16)] = z


def _zero_stripe(acc, buf, row0, n_full, tail, width):
  @pl.loop(0, n_full)
  def _chunk(i):
    pltpu.sync_copy(buf, acc.at[pl.ds(row0 + i * 128, 128)])
  if tail:
    pltpu.sync_copy(buf.at[pl.ds(0, tail)],
                    acc.at[pl.ds(row0 + n_full * 128, tail)])

_MESH = plsc.VectorSubcoreMesh(
    core_axis_name="c", subcore_axis_name="s", num_cores=NC, num_subcores=NS
)


_GATHER_DNUMS = lax.GatherDimensionNumbers(
    offset_dims=(), collapsed_slice_dims=(0,), start_index_map=(0,))


def _lane_bcast(v16, i):
  idx = jnp.broadcast_to(i, (16,)).astype(jnp.int32)
  return lax.gather(v16, idx[:, None], _GATHER_DNUMS, (1,),
                    mode=lax.GatherScatterMode.PROMISE_IN_BOUNDS)


def _scale_rows(rows_v, w_v, j, width):
  @pl.loop(0, 8)
  def _grp(l):
    w16 = w_v[j, pl.ds(l * 16, 16)]
    for t in range(16):
      ws = _lane_bcast(w16, t)
      e = l * 16 + t
      for k in range(width // 16):
        sl = pl.ds(k * 16, 16)
        rows_v[e, sl] = rows_v[e, sl] * ws


GG = 16


def _edge_pipeline(x_hbm, src_hbm, dst_hbm, w_hbm, base, n_groups,
                   src_v, dst_v, w_v, rows_a, rows_b, acc,
                   sem_a, sem_b, ssem_a, ssem_b, width):
  del ssem_a, ssem_b

  def gather(j, buf, sem):
    pltpu.async_copy(x_hbm.at[src_v.at[j]], buf, sem)

  def consume(j, buf, sem):
    pltpu.make_async_copy(x_hbm.at[src_v.at[j]], buf, sem).wait()
    _scale_rows(buf, w_v, j, width)
    pltpu.sync_copy(buf, acc.at[dst_v.at[j]], add=True)

  @pl.loop(0, n_groups)
  def _group(g):
    row0 = base + g * GG
    pltpu.sync_copy(src_hbm.at[pl.ds(row0, GG)], src_v)
    pltpu.sync_copy(dst_hbm.at[pl.ds(row0, GG)], dst_v)
    pltpu.sync_copy(w_hbm.at[pl.ds(row0, GG)], w_v)
    gather(0, rows_a, sem_a)

    @pl.loop(0, GG // 2)
    def _pair(m):
      ja = 2 * m
      jb = 2 * m + 1
      gather(jb, rows_b, sem_b)
      consume(ja, rows_a, sem_a)

      @pl.when(m < GG // 2 - 1)
      def _():
        gather(jb + 1, rows_a, sem_a)
      consume(jb, rows_b, sem_b)


def _sc_u2i_body(x_hbm, src_hbm, dst_hbm, w_hbm, out_hbm,
                 src_v, dst_v, w_v, rows_a, rows_b, acc,
                 sem_a, sem_b, ssem_a, ssem_b):
  c = lax.axis_index("c")
  s = lax.axis_index("s")
  stripe = N_ITEM_PAD // NS
  base, n_groups = _core_base_rows(c, s, U2I_SPLIT)
  _memset_zero(rows_b, D)
  _zero_stripe(acc, rows_b, s * stripe, stripe // 128, 0, D)
  plsc.subcore_barrier()

  _edge_pipeline(x_hbm, src_hbm, dst_hbm, w_hbm, base, n_groups,
                 src_v, dst_v, w_v, rows_a, rows_b, acc,
                 sem_a, sem_b, ssem_a, ssem_b, D)

  plsc.subcore_barrier()
  pltpu.sync_copy(acc.at[pl.ds(s * stripe, stripe)],
                  out_hbm.at[c].at[pl.ds(s * stripe, stripe)])


_sc_u2i = functools.partial(
    pl.kernel,
    out_type=jax.ShapeDtypeStruct((NC, N_ITEM_PAD, D), jnp.float32),
    mesh=_MESH,
    scratch_types=[
        pltpu.VMEM((GG, 128), jnp.int32),
        pltpu.VMEM((GG, 128), jnp.int32),
        pltpu.VMEM((GG, 128), jnp.float32),
        pltpu.VMEM((128, D), jnp.float32),
        pltpu.VMEM((128, D), jnp.float32),
        pltpu.VMEM_SHARED((N_ITEM_PAD, D), jnp.float32),
        pltpu.SemaphoreType.DMA,
        pltpu.SemaphoreType.DMA,
        pltpu.SemaphoreType.DMA,
        pltpu.SemaphoreType.DMA,
    ],
)(_sc_u2i_body)


def _sc_i2u_body(x4_hbm, src_hbm, dst_hbm, w_hbm, out_hbm,
                 src_v, dst_v, w_v, rows_a, rows_b, acc,
                 sem_a, sem_b, ssem_a, ssem_b):
  c = lax.axis_index("c")
  s = lax.axis_index("s")
  stripe = N_USER_PAD // NS
  base, n_groups = _core_base_rows(c, s, I2U_SPLIT)

  for q in range(4):
    _memset_zero(rows_b, 32)
    _zero_stripe(acc, rows_b, s * stripe, stripe // 128, stripe % 128, 32)
    plsc.subcore_barrier()

    _edge_pipeline(x4_hbm.at[q], src_hbm, dst_hbm, w_hbm, base, n_groups,
                   src_v, dst_v, w_v, rows_a, rows_b, acc,
                   sem_a, sem_b, ssem_a, ssem_b, 32)

    plsc.subcore_barrier()
    pltpu.sync_copy(acc.at[pl.ds(s * stripe, stripe)],
                    out_hbm.at[c].at[pl.ds(s * stripe, stripe),
                                     pl.ds(q * 32, 32)])
    plsc.subcore_barrier()


_sc_i2u = functools.partial(
    pl.kernel,
    out_type=jax.ShapeDtypeStruct((NC, N_USER_PAD, D), jnp.float32),
    mesh=_MESH,
    scratch_types=[
        pltpu.VMEM((GG, 128), jnp.int32),
        pltpu.VMEM((GG, 128), jnp.int32),
        pltpu.VMEM((GG, 128), jnp.float32),
        pltpu.VMEM((128, 32), jnp.float32),
        pltpu.VMEM((128, 32), jnp.float32),
        pltpu.VMEM_SHARED((N_USER_PAD, 32), jnp.float32),
        pltpu.SemaphoreType.DMA,
        pltpu.SemaphoreType.DMA,
        pltpu.SemaphoreType.DMA,
        pltpu.SemaphoreType.DMA,
    ],
    compiler_params=pltpu.CompilerParams(use_tc_tiling_on_sc=False),
)(_sc_i2u_body)


def _tc_combine_body(p_ref, x_ref, wm_ref, ws_ref, b_ref, o_ref):
  if p_ref.shape[0] == 2:
    agg = p_ref[0] + p_ref[1]
  else:
    agg = p_ref[0]
  o_ref[...] = (
      jnp.dot(agg, wm_ref[...], preferred_element_type=jnp.float32)
      + jnp.dot(x_ref[...], ws_ref[...], preferred_element_type=jnp.float32)
      + b_ref[...]
  )


def _tc_combine(p, x, wm, ws, b, blk):
  n = x.shape[0]
  if p.ndim == 2:
    p = p.reshape(1, *p.shape)
  nparts = p.shape[0]
  return pl.pallas_call(
      _tc_combine_body,
      grid=(n // blk,),
      in_specs=[
          pl.BlockSpec((nparts, blk, D), lambda i: (0, i, 0)),
          pl.BlockSpec((blk, D), lambda i: (i, 0)),
          pl.BlockSpec((D, D), lambda i: (0, 0)),
          pl.BlockSpec((D, D), lambda i: (0, 0)),
          pl.BlockSpec((1, D), lambda i: (0, 0)),
      ],
      out_specs=pl.BlockSpec((blk, D), lambda i: (i, 0)),
      out_shape=jax.ShapeDtypeStruct((n, D), jnp.float32),
  )(p, x, wm, ws, b.reshape(1, D))


def _prep_edges(src, dst, w):
  pad = E_PAD - E
  src = jnp.concatenate([src.astype(jnp.int32), jnp.zeros((pad,), jnp.int32)])
  dst = jnp.concatenate([dst.astype(jnp.int32), jnp.zeros((pad,), jnp.int32)])
  w = jnp.concatenate([w, jnp.zeros((pad,), jnp.float32)])
  return src.reshape(-1, 128), dst.reshape(-1, 128), w.reshape(-1, 128)


def kernel(x_user, x_item, src_u2i, dst_u2i, edge_weight_u2i,
           src_i2u, dst_i2u, edge_weight_i2u,
           W_msg_u2i, W_self_u2i, b_u2i,
           W_msg_i2u, W_self_i2u, b_i2u):
  su, du, wu = _prep_edges(src_u2i, dst_u2i, edge_weight_u2i)
  si, di, wi = _prep_edges(src_i2u, dst_i2u, edge_weight_i2u)
  xi4 = x_item.reshape(N_ITEM, 4, 32).transpose(1, 0, 2)

  p_item = _sc_u2i(x_user, su, du, wu)
  p_user = _sc_i2u(xi4, si, di, wi)

  out_item = _tc_combine(p_item, x_item, W_msg_u2i, W_self_u2i, b_u2i, 2000)
  out_user = _tc_combine(p_user, x_user, W_msg_i2u, W_self_i2u, b_i2u, 2000)
  return (out_user, out_item)
